# Initial kernel scaffold; baseline (speedup 1.0000x reference)
import jax
import jax.numpy as jnp
from jax.experimental import pallas as pl


def kernel(x, edge_index, W1, b1, W2, b2, g1, beta1, W3, b3, W4, b4, g2, beta2):
    raise NotImplementedError("write your pallas kernel here")


# speedup vs baseline: 11.1980x; 11.1980x over previous
import functools

import jax
import jax.numpy as jnp
from jax import lax
from jax.experimental import pallas as pl
from jax.experimental.pallas import tpu as pltpu
from jax.experimental.pallas import tpu_sc as plsc

N = 10000
F_IN = 128
DIM = 16
C = 128

NC = 2
NS = 16
NW = NC * NS
CHUNK = 128
N_PAD = 10112


def _sc_agg_body(y_hbm, zeros_hbm, src_hbm, dst_hbm, out_hbm,
                 idx_s_v, idx_d_v, rows_v, acc_sh, sem):
    c = lax.axis_index("c")
    s = lax.axis_index("s")
    rows_per_sub = N_PAD // NS
    sl = pl.ds(s * rows_per_sub, rows_per_sub)
    pltpu.sync_copy(zeros_hbm.at[sl], acc_sh.at[sl])
    plsc.subcore_barrier()

    wid = s * NC + c
    pltpu.sync_copy(src_hbm.at[wid], idx_s_v)
    pltpu.sync_copy(dst_hbm.at[wid], idx_d_v)

    n_chunks = src_hbm.shape[1]

    def chunk_body(j, carry):
        pltpu.async_copy(y_hbm.at[idx_s_v.at[j]], rows_v, sem).wait()
        pltpu.sync_copy(rows_v, acc_sh.at[idx_d_v.at[j]], add=True)
        return carry

    lax.fori_loop(0, n_chunks, chunk_body, 0)
    plsc.subcore_barrier()
    pltpu.sync_copy(acc_sh.at[sl], out_hbm.at[c, sl])


@jax.jit
def _sc_agg(y_pad, zeros_pad, src_p, dst_p):
    n_chunks = src_p.shape[1]
    mesh = plsc.VectorSubcoreMesh(core_axis_name="c", subcore_axis_name="s",
                                  num_cores=NC, num_subcores=NS)
    return pl.kernel(
        _sc_agg_body,
        out_type=jax.ShapeDtypeStruct((NC, N_PAD, DIM), jnp.float32),
        mesh=mesh,
        scratch_types=[
            pltpu.VMEM((n_chunks, CHUNK), jnp.int32),
            pltpu.VMEM((n_chunks, CHUNK), jnp.int32),
            pltpu.VMEM((CHUNK, DIM), jnp.float32),
            pltpu.MemorySpace.VMEM_SHARED((N_PAD, DIM), jnp.float32),
            pltpu.SemaphoreType.DMA,
        ],
        compiler_params=pltpu.CompilerParams(use_tc_tiling_on_sc=False),
    )(y_pad, zeros_pad, src_p, dst_p)


def _mm1_body(x_ref, w_ref, o_ref):
    o_ref[0:N] = jnp.dot(x_ref[...], w_ref[...],
                         preferred_element_type=jnp.float32, precision=lax.Precision.HIGHEST)
    o_ref[N:] = jnp.zeros((N_PAD - N, DIM), jnp.float32)


@jax.jit
def _mm1(x, W1):
    return pl.pallas_call(
        _mm1_body,
        out_shape=jax.ShapeDtypeStruct((N_PAD, DIM), jnp.float32),
    )(x, W1)


def _bn(m, g, beta):
    mean = jnp.mean(m, axis=0, keepdims=True)
    var = jnp.mean(jnp.square(m - mean), axis=0, keepdims=True)
    return (m - mean) / jnp.sqrt(var + 1e-5) * g + beta


def _mlp1_body(p_ref, y_ref, b1_ref, w2_ref, b2_ref, g1_ref, bt1_ref, o_ref):
    z = p_ref[0, 0:N, :] + p_ref[1, 0:N, :] + y_ref[0:N]
    a = jnp.maximum(z + b1_ref[...], 0.0)
    m = jnp.dot(a, w2_ref[...], preferred_element_type=jnp.float32) + b2_ref[...]
    m = jnp.maximum(m, 0.0)
    o_ref[0:N] = _bn(m, g1_ref[...], bt1_ref[...])
    o_ref[N:] = jnp.zeros((N_PAD - N, DIM), jnp.float32)


@jax.jit
def _mlp1(p, y1, b1, W2, b2, g1, beta1):
    return pl.pallas_call(
        _mlp1_body,
        out_shape=jax.ShapeDtypeStruct((N_PAD, DIM), jnp.float32),
    )(p, y1, b1, W2, b2, g1, beta1)


def _mlp2_body(p_ref, h_ref, w3_ref, b3_ref, w4_ref, b4_ref, g2_ref, bt2_ref,
               lp_ref, o_ref):
    z = p_ref[0, 0:N, :] + p_ref[1, 0:N, :] + h_ref[0:N]
    t = jnp.maximum(jnp.dot(z, w3_ref[...],
                            preferred_element_type=jnp.float32) + b3_ref[...], 0.0)
    o = jnp.dot(t, w4_ref[...], preferred_element_type=jnp.float32) + b4_ref[...]
    o = jnp.maximum(o, 0.0)
    o = _bn(o, g2_ref[...], bt2_ref[...])
    mx = jnp.max(o, axis=1, keepdims=True)
    lse = jnp.log(jnp.sum(jnp.exp(o - mx), axis=1, keepdims=True)) + mx
    lp_ref[...] = o - lse
    o_ref[...] = o


@jax.jit
def _mlp2(p, h, W3, b3, W4, b4, g2, beta2):
    return pl.pallas_call(
        _mlp2_body,
        out_shape=(
            jax.ShapeDtypeStruct((N, C), jnp.float32),
            jax.ShapeDtypeStruct((N, C), jnp.float32),
        ),
    )(p, h, W3, b3, W4, b4, g2, beta2)


def kernel(x, edge_index, W1, b1, W2, b2, g1, beta1, W3, b3, W4, b4, g2, beta2):
    src = edge_index[0]
    dst = edge_index[1]
    E = src.shape[0]
    n_chunks = -(-E // (NW * CHUNK))
    E_pad = NW * n_chunks * CHUNK
    src_p = jnp.concatenate(
        [src, jnp.zeros((E_pad - E,), jnp.int32)]).reshape(NW, n_chunks, CHUNK)
    dst_p = jnp.concatenate(
        [dst, jnp.full((E_pad - E,), N, jnp.int32)]).reshape(NW, n_chunks, CHUNK)
    zeros_pad = jnp.zeros((N_PAD, DIM), jnp.float32)

    b1r = b1.reshape(1, DIM)
    b2r = b2.reshape(1, DIM)
    b3r = b3.reshape(1, DIM)
    b4r = b4.reshape(1, C)
    g1r = g1.reshape(1, DIM)
    bt1r = beta1.reshape(1, DIM)
    g2r = g2.reshape(1, C)
    bt2r = beta2.reshape(1, C)

    y1 = _mm1(x, W1)
    p1 = _sc_agg(y1, zeros_pad, src_p, dst_p)
    h = _mlp1(p1, y1, b1r, W2, b2r, g1r, bt1r)
    p2 = _sc_agg(h, zeros_pad, src_p, dst_p)
    lp, out = _mlp2(p2, h, W3, b3r, W4, b4r, g2r, bt2r)
    return (lp, out)
